# baseline (speedup 1.0000x reference)
import functools

import jax
import jax.numpy as jnp
from jax import lax
from jax.experimental import pallas as pl
from jax.experimental.pallas import tpu as pltpu, tpu_sc as plsc

N = 10000
E = 320000
D = 128
KAPPA = 0.95

NC = 2
NS = 16
NW = NC * NS
CHUNK = 128
CHUNKS = 80
E_PAD = NW * CHUNKS * CHUNK
N_PAD = 10240
ROWS_PER_TILE = N_PAD // NS

_mesh = plsc.VectorSubcoreMesh(core_axis_name="c", subcore_axis_name="s")


@functools.partial(
    pl.kernel,
    out_type=jax.ShapeDtypeStruct((NC, N_PAD), jnp.float32),
    mesh=_mesh,
    scratch_types=[
        pltpu.VMEM((CHUNKS, CHUNK), jnp.int32),
        pltpu.VMEM((CHUNK,), jnp.float32),
        pltpu.VMEM((ROWS_PER_TILE,), jnp.float32),
        pltpu.VMEM_SHARED((N_PAD,), jnp.float32),
        pltpu.SemaphoreType.DMA,
    ],
)
def _deg_kernel(dst_hbm, deg_out, idx_v, ones_v, stage_v, deg_sh, sem):
    cid = lax.axis_index("c")
    sid = lax.axis_index("s")
    wid = sid * NC + cid
    r0 = sid * ROWS_PER_TILE

    for k in range(CHUNK // 16):
        ones_v[pl.ds(k * 16, 16)] = jnp.ones((16,), jnp.float32)
    for k in range(ROWS_PER_TILE // 16):
        stage_v[pl.ds(k * 16, 16)] = jnp.zeros((16,), jnp.float32)
    pltpu.sync_copy(stage_v, deg_sh.at[pl.ds(r0, ROWS_PER_TILE)])
    pltpu.sync_copy(dst_hbm.at[wid], idx_v)
    plsc.subcore_barrier()

    def body(j, carry):
        pltpu.async_copy(ones_v, deg_sh.at[idx_v.at[j]], sem, add=True)
        return carry

    lax.fori_loop(0, CHUNKS, body, 0)

    def drain(j, carry):
        pltpu.make_async_copy(ones_v, deg_sh.at[idx_v.at[j]], sem).wait()
        return carry

    lax.fori_loop(0, CHUNKS, drain, 0)
    plsc.subcore_barrier()
    pltpu.sync_copy(deg_sh.at[pl.ds(r0, ROWS_PER_TILE)],
                    deg_out.at[cid, pl.ds(r0, ROWS_PER_TILE)])


@functools.partial(
    pl.kernel,
    out_type=jax.ShapeDtypeStruct((NC, N_PAD, D), jnp.float32),
    mesh=_mesh,
    scratch_types=[
        pltpu.VMEM((CHUNKS // 2, CHUNK), jnp.int32),
        pltpu.VMEM((CHUNKS // 2, CHUNK), jnp.int32),
        pltpu.VMEM((CHUNK, D), jnp.float32),
        pltpu.VMEM((CHUNK, D), jnp.float32),
        pltpu.VMEM_SHARED((N_PAD, D), jnp.float32),
        pltpu.SemaphoreType.DMA,
        pltpu.SemaphoreType.DMA,
        pltpu.SemaphoreType.DMA,
    ],
)
def _scatter_kernel(src_hbm, dst_hbm, y_hbm, acc_out,
                    src_v, dst_v, buf_a, buf_b, acc_sh, sem_a, sem_b, sem_i):
    cid = lax.axis_index("c")
    sid = lax.axis_index("s")
    wid = sid * NC + cid
    r0 = sid * ROWS_PER_TILE
    half = CHUNKS // 2

    init = pltpu.async_copy(y_hbm.at[pl.ds(r0, ROWS_PER_TILE)],
                            acc_sh.at[pl.ds(r0, ROWS_PER_TILE)], sem_i)
    pltpu.sync_copy(src_hbm.at[wid, pl.ds(0, half)], src_v)
    pltpu.sync_copy(dst_hbm.at[wid, pl.ds(0, half)], dst_v)
    pltpu.async_copy(y_hbm.at[src_v.at[0]], buf_a, sem_a)
    init.wait()
    plsc.subcore_barrier()

    for phase in range(2):
        if phase:
            pltpu.sync_copy(src_hbm.at[wid, pl.ds(half, half)], src_v)
            pltpu.sync_copy(dst_hbm.at[wid, pl.ds(half, half)], dst_v)
            pltpu.async_copy(y_hbm.at[src_v.at[0]], buf_a, sem_a)

        def body(g, carry):
            j = 2 * g
            pltpu.async_copy(y_hbm.at[src_v.at[j + 1]], buf_b, sem_b)
            pltpu.make_async_copy(y_hbm.at[src_v.at[j]], buf_a, sem_a).wait()
            pltpu.sync_copy(buf_a, acc_sh.at[dst_v.at[j]], add=True)

            @pl.when(g < half // 2 - 1)
            def _():
                pltpu.async_copy(y_hbm.at[src_v.at[j + 2]], buf_a, sem_a)

            pltpu.make_async_copy(y_hbm.at[src_v.at[j + 1]], buf_b,
                                  sem_b).wait()
            pltpu.sync_copy(buf_b, acc_sh.at[dst_v.at[j + 1]], add=True)
            return carry

        lax.fori_loop(0, half // 2, body, 0)
    plsc.subcore_barrier()
    pltpu.sync_copy(acc_sh.at[pl.ds(r0, ROWS_PER_TILE)],
                    acc_out.at[cid, pl.ds(r0, ROWS_PER_TILE)])


def _dense_a_body(wc_ref, emb_ref, feat_ref, wmlp_ref, xw_ref, z_ref):
    A = wc_ref[...]
    absA = jnp.abs(A)
    row_sum = jnp.sum(absA, axis=1, keepdims=True)
    hi0 = jnp.max(absA, axis=1, keepdims=True)

    def body(i, carry):
        lo, hi = carry
        th = (lo + hi) * 0.5
        s = jnp.sum(jnp.maximum(absA - th, 0.0), axis=1, keepdims=True)
        pred = s > KAPPA
        return jnp.where(pred, th, lo), jnp.where(pred, hi, th)

    lo, hi = lax.fori_loop(0, 40, body, (jnp.zeros_like(hi0), hi0))
    th = (lo + hi) * 0.5
    proj = jnp.sign(A) * jnp.maximum(absA - th, 0.0)
    Wc = jnp.where(row_sum > KAPPA, proj, A)

    xw_ref[...] = jnp.dot(emb_ref[...], Wc.T, preferred_element_type=jnp.float32)
    z_ref[...] = jnp.dot(feat_ref[...], wmlp_ref[...].T,
                         preferred_element_type=jnp.float32)


_dense_a = pl.pallas_call(
    _dense_a_body,
    out_shape=(
        jax.ShapeDtypeStruct((N, D), jnp.float32),
        jax.ShapeDtypeStruct((N, D), jnp.float32),
    ),
)


def _dense_b_body(xw_ref, deg_ref, y_ref, dinv_ref):
    deg = deg_ref[0, :] + deg_ref[1, :] + 1.0
    dinv = lax.rsqrt(deg).reshape(N_PAD, 1)
    dinv_ref[...] = dinv
    y_ref[pl.ds(0, N), :] = xw_ref[...] * dinv[:N]
    y_ref[pl.ds(N, N_PAD - N), :] = jnp.zeros((N_PAD - N, D), jnp.float32)


_dense_b = pl.pallas_call(
    _dense_b_body,
    out_shape=(
        jax.ShapeDtypeStruct((N_PAD, D), jnp.float32),
        jax.ShapeDtypeStruct((N_PAD, 1), jnp.float32),
    ),
)


def _final_body(acc_ref, y_ref, dinv_ref, z_ref, o_ref):
    a = acc_ref[0, pl.ds(0, N), :] + acc_ref[1, pl.ds(0, N), :] \
        - y_ref[pl.ds(0, N), :]
    o_ref[...] = jnp.maximum(a * dinv_ref[pl.ds(0, N)] + z_ref[...], 0.0)


_final = pl.pallas_call(
    _final_body,
    out_shape=jax.ShapeDtypeStruct((N, D), jnp.float32),
)


def kernel(features, sparse_adj, embeddings, W_conv, W_mlp):
    src = sparse_adj[0]
    dst = sparse_adj[1]
    pad = N + (jnp.arange(E_PAD - E, dtype=jnp.int32) & 127)
    src_p = jnp.concatenate([src, pad]).reshape(NW, CHUNKS, CHUNK)
    dst_p = jnp.concatenate([dst, pad]).reshape(NW, CHUNKS, CHUNK)

    deg_parts = _deg_kernel(dst_p)
    xw, z = _dense_a(W_conv, embeddings, features, W_mlp)
    y, dinv = _dense_b(xw, deg_parts)
    acc = _scatter_kernel(src_p, dst_p, y)
    return _final(acc, y, dinv, z)


# speedup vs baseline: 1.0042x; 1.0042x over previous
import functools

import jax
import jax.numpy as jnp
from jax import lax
from jax.experimental import pallas as pl
from jax.experimental.pallas import tpu as pltpu, tpu_sc as plsc

N = 10000
E = 320000
D = 128
KAPPA = 0.95

NC = 2
NS = 16
NW = NC * NS
CHUNK = 128
CHUNKS = 80
E_PAD = NW * CHUNKS * CHUNK
N_PAD = 10240
ROWS_PER_TILE = N_PAD // NS

_mesh = plsc.VectorSubcoreMesh(core_axis_name="c", subcore_axis_name="s")


@functools.partial(
    pl.kernel,
    out_type=jax.ShapeDtypeStruct((NC, N_PAD), jnp.float32),
    mesh=_mesh,
    scratch_types=[
        pltpu.VMEM((CHUNKS, CHUNK), jnp.int32),
        pltpu.VMEM((CHUNK,), jnp.float32),
        pltpu.VMEM((ROWS_PER_TILE,), jnp.float32),
        pltpu.VMEM_SHARED((N_PAD,), jnp.float32),
        pltpu.SemaphoreType.DMA,
    ],
)
def _deg_kernel(dst_hbm, deg_out, idx_v, ones_v, stage_v, deg_sh, sem):
    cid = lax.axis_index("c")
    sid = lax.axis_index("s")
    wid = sid * NC + cid
    r0 = sid * ROWS_PER_TILE

    for k in range(CHUNK // 16):
        ones_v[pl.ds(k * 16, 16)] = jnp.ones((16,), jnp.float32)
    for k in range(ROWS_PER_TILE // 16):
        stage_v[pl.ds(k * 16, 16)] = jnp.zeros((16,), jnp.float32)
    pltpu.sync_copy(stage_v, deg_sh.at[pl.ds(r0, ROWS_PER_TILE)])
    pltpu.sync_copy(dst_hbm.at[wid], idx_v)
    plsc.subcore_barrier()

    def body(j, carry):
        pltpu.async_copy(ones_v, deg_sh.at[idx_v.at[j]], sem, add=True)
        return carry

    lax.fori_loop(0, CHUNKS, body, 0)

    def drain(j, carry):
        pltpu.make_async_copy(ones_v, deg_sh.at[idx_v.at[j]], sem).wait()
        return carry

    lax.fori_loop(0, CHUNKS, drain, 0)
    plsc.subcore_barrier()
    pltpu.sync_copy(deg_sh.at[pl.ds(r0, ROWS_PER_TILE)],
                    deg_out.at[cid, pl.ds(r0, ROWS_PER_TILE)])


@functools.partial(
    pl.kernel,
    out_type=jax.ShapeDtypeStruct((NC, N_PAD, D), jnp.float32),
    mesh=_mesh,
    scratch_types=[
        pltpu.VMEM((CHUNKS // 2, CHUNK), jnp.int32),
        pltpu.VMEM((CHUNKS // 2, CHUNK), jnp.int32),
        pltpu.VMEM((CHUNK, D), jnp.float32),
        pltpu.VMEM((CHUNK, D), jnp.float32),
        pltpu.VMEM_SHARED((N_PAD, D), jnp.float32),
        pltpu.SemaphoreType.DMA,
        pltpu.SemaphoreType.DMA,
        pltpu.SemaphoreType.DMA,
    ],
)
def _scatter_kernel(src_hbm, dst_hbm, y_hbm, acc_out,
                    src_v, dst_v, buf_a, buf_b, acc_sh, sem_a, sem_b, sem_i):
    cid = lax.axis_index("c")
    sid = lax.axis_index("s")
    wid = sid * NC + cid
    r0 = sid * ROWS_PER_TILE
    half = CHUNKS // 2

    init = pltpu.async_copy(y_hbm.at[pl.ds(r0, ROWS_PER_TILE)],
                            acc_sh.at[pl.ds(r0, ROWS_PER_TILE)], sem_i)
    pltpu.sync_copy(src_hbm.at[wid, pl.ds(0, half)], src_v)
    pltpu.sync_copy(dst_hbm.at[wid, pl.ds(0, half)], dst_v)
    pltpu.async_copy(y_hbm.at[src_v.at[0]], buf_a, sem_a)
    init.wait()
    plsc.subcore_barrier()

    for phase in range(2):
        if phase:
            pltpu.sync_copy(src_hbm.at[wid, pl.ds(half, half)], src_v)
            pltpu.sync_copy(dst_hbm.at[wid, pl.ds(half, half)], dst_v)
            pltpu.async_copy(y_hbm.at[src_v.at[0]], buf_a, sem_a)

        def body(g, carry):
            j = 2 * g
            pltpu.async_copy(y_hbm.at[src_v.at[j + 1]], buf_b, sem_b)
            pltpu.make_async_copy(y_hbm.at[src_v.at[j]], buf_a, sem_a).wait()
            pltpu.sync_copy(buf_a, acc_sh.at[dst_v.at[j]], add=True)

            @pl.when(g < half // 2 - 1)
            def _():
                pltpu.async_copy(y_hbm.at[src_v.at[j + 2]], buf_a, sem_a)

            pltpu.make_async_copy(y_hbm.at[src_v.at[j + 1]], buf_b,
                                  sem_b).wait()
            pltpu.sync_copy(buf_b, acc_sh.at[dst_v.at[j + 1]], add=True)
            return carry

        lax.fori_loop(0, half // 2, body, 0)
    plsc.subcore_barrier()
    pltpu.sync_copy(acc_sh.at[pl.ds(r0, ROWS_PER_TILE)],
                    acc_out.at[cid, pl.ds(r0, ROWS_PER_TILE)])


def _dense_a_body(wc_ref, emb_ref, feat_ref, wmlp_ref, xw_ref, z_ref):
    A = wc_ref[...]
    absA = jnp.abs(A)
    row_sum = jnp.sum(absA, axis=1, keepdims=True)
    hi0 = jnp.max(absA, axis=1, keepdims=True)

    def body(i, carry):
        lo, hi = carry
        th = (lo + hi) * 0.5
        s = jnp.sum(jnp.maximum(absA - th, 0.0), axis=1, keepdims=True)
        pred = s > KAPPA
        return jnp.where(pred, th, lo), jnp.where(pred, hi, th)

    lo, hi = lax.fori_loop(0, 40, body, (jnp.zeros_like(hi0), hi0))
    th = (lo + hi) * 0.5
    proj = jnp.sign(A) * jnp.maximum(absA - th, 0.0)
    Wc = jnp.where(row_sum > KAPPA, proj, A)

    xw_ref[...] = jnp.dot(emb_ref[...], Wc.T, preferred_element_type=jnp.float32)
    z_ref[...] = jnp.dot(feat_ref[...], wmlp_ref[...].T,
                         preferred_element_type=jnp.float32)


_dense_a = pl.pallas_call(
    _dense_a_body,
    out_shape=(
        jax.ShapeDtypeStruct((N, D), jnp.float32),
        jax.ShapeDtypeStruct((N, D), jnp.float32),
    ),
)


def _dense_b_body(xw_ref, deg_ref, y_ref, dinv_ref):
    deg = deg_ref[0, :] + deg_ref[1, :] + 1.0
    dinv = lax.rsqrt(deg).reshape(N_PAD, 1)
    dinv_ref[...] = dinv
    y_ref[pl.ds(0, N), :] = xw_ref[...] * dinv[:N]
    y_ref[pl.ds(N, N_PAD - N), :] = jnp.zeros((N_PAD - N, D), jnp.float32)


_dense_b = pl.pallas_call(
    _dense_b_body,
    out_shape=(
        jax.ShapeDtypeStruct((N_PAD, D), jnp.float32),
        jax.ShapeDtypeStruct((N_PAD, 1), jnp.float32),
    ),
)


def _final_body(acc_ref, y_ref, dinv_ref, z_ref, o_ref):
    a = acc_ref[0] + acc_ref[1] - y_ref[...]
    o_ref[...] = jnp.maximum(a * dinv_ref[...] + z_ref[...], 0.0)


_FB = 2000

_final = pl.pallas_call(
    _final_body,
    grid=(N // _FB,),
    in_specs=[
        pl.BlockSpec((NC, _FB, D), lambda i: (0, i, 0)),
        pl.BlockSpec((_FB, D), lambda i: (i, 0)),
        pl.BlockSpec((_FB, 1), lambda i: (i, 0)),
        pl.BlockSpec((_FB, D), lambda i: (i, 0)),
    ],
    out_specs=pl.BlockSpec((_FB, D), lambda i: (i, 0)),
    out_shape=jax.ShapeDtypeStruct((N, D), jnp.float32),
)


def kernel(features, sparse_adj, embeddings, W_conv, W_mlp):
    src = sparse_adj[0]
    dst = sparse_adj[1]
    pad = N + (jnp.arange(E_PAD - E, dtype=jnp.int32) & 127)
    src_p = jnp.concatenate([src, pad]).reshape(NW, CHUNKS, CHUNK)
    dst_p = jnp.concatenate([dst, pad]).reshape(NW, CHUNKS, CHUNK)

    deg_parts = _deg_kernel(dst_p)
    xw, z = _dense_a(W_conv, embeddings, features, W_mlp)
    y, dinv = _dense_b(xw, deg_parts)
    acc = _scatter_kernel(src_p, dst_p, y)
    return _final(acc, y, dinv, z)
